# Initial kernel scaffold; baseline (speedup 1.0000x reference)
import jax
import jax.numpy as jnp
from jax.experimental import pallas as pl


def kernel(x, edge_index, W1, b1, W2, b2):
    raise NotImplementedError("write your pallas kernel here")


# speedup vs baseline: 30.8207x; 30.8207x over previous
import functools

import jax
import jax.numpy as jnp
from jax import lax
from jax.experimental import pallas as pl
from jax.experimental.pallas import tpu as pltpu
from jax.experimental.pallas import tpu_sc as plsc

N = 10000
E = 320000
D = 128
NC = 2
NS = 16
NW = NC * NS
K = 128
NB_AGG = 80
PH = 2
NB_PH = NB_AGG // PH
NB_DEG = 160
E_PAD = NW * NB_AGG * K
NACC = 10112
RPT = NACC // NS
NBUF = 2
ROWB = 1000


def _mesh():
    return plsc.VectorSubcoreMesh(
        core_axis_name="c", subcore_axis_name="s", num_cores=NC, num_subcores=NS
    )


def _splat16(x):
    return jnp.full((16,), x, dtype=jnp.int32)


def _deg_body(dst_hbm, deg_out_hbm, acc, dst_v, ones_v, zbuf, deg_v, out_v):
    c = lax.axis_index("c")
    s = lax.axis_index("s")

    @pl.when(c == 0)
    def _():
        one = jnp.ones((16,), jnp.float32)
        zero = jnp.zeros((16,), jnp.float32)
        for i in range(K // 16):
            ones_v[pl.ds(16 * i, 16)] = one
        for i in range(RPT // 16):
            zbuf[pl.ds(16 * i, 16)] = zero
        zbuf[pl.ds(RPT - 16, 16)] = zero
        pltpu.sync_copy(zbuf, acc.at[pl.ds(RPT * s, RPT)])
        pltpu.sync_copy(dst_hbm.at[s], dst_v)
        plsc.subcore_barrier()

        def scat(j, carry):
            pltpu.sync_copy(ones_v, acc.at[dst_v.at[j]], add=True)
            return carry

        lax.fori_loop(0, NB_DEG, scat, 0)
        plsc.subcore_barrier()

        pltpu.sync_copy(acc.at[pl.ds(RPT * s, RPT)], deg_v)

        lanes = lax.broadcasted_iota(jnp.int32, (16,), 0)

        def rep(r, carry):
            sp = plsc.load_gather(deg_v, [_splat16(r)])
            for k8 in range(D // 16):
                plsc.store_scatter(out_v, [_splat16(r), lanes + 16 * k8], sp)
            return carry

        lax.fori_loop(0, RPT, rep, 0)
        pltpu.sync_copy(out_v, deg_out_hbm.at[pl.ds(RPT * s, RPT)])


@functools.cache
def _deg_kernel():
    return pl.kernel(
    _deg_body,
    out_type=jax.ShapeDtypeStruct((NACC, D), jnp.float32),
    mesh=_mesh(),
    compiler_params=pltpu.CompilerParams(needs_layout_passes=False),
    scratch_types=[
        pltpu.VMEM_SHARED((NACC,), jnp.float32),
        pltpu.VMEM((NB_DEG, K), jnp.int32),
        pltpu.VMEM((K,), jnp.float32),
        pltpu.VMEM((RPT,), jnp.float32),
        pltpu.VMEM((RPT,), jnp.float32),
        pltpu.VMEM((RPT, D), jnp.float32),
    ],
    )


def _agg_body(y_hbm, src_hbm, dst_hbm, zeros_hbm, out_hbm,
              acc, src_v, dst_v, bufs, sems):
    c = lax.axis_index("c")
    s = lax.axis_index("s")
    w = c * NS + s

    pltpu.sync_copy(zeros_hbm.at[pl.ds(RPT * s, RPT)], acc.at[pl.ds(RPT * s, RPT)])
    plsc.subcore_barrier()

    for ph in range(PH):
        pltpu.sync_copy(src_hbm.at[w, pl.ds(ph * NB_PH, NB_PH)], src_v)
        pltpu.sync_copy(dst_hbm.at[w, pl.ds(ph * NB_PH, NB_PH)], dst_v)

        for b in range(NBUF):
            pltpu.async_copy(y_hbm.at[src_v.at[b]], bufs[b], sems[b])

        def outer(j, carry):
            for b in range(NBUF):
                jj = j * NBUF + b
                pltpu.make_async_copy(
                    y_hbm.at[src_v.at[jj]], bufs[b], sems[b]).wait()
                pltpu.sync_copy(bufs[b], acc.at[dst_v.at[jj]], add=True)

                @pl.when(jj + NBUF < NB_PH)
                def _():
                    pltpu.async_copy(
                        y_hbm.at[src_v.at[jj + NBUF]], bufs[b], sems[b])

            return carry

        lax.fori_loop(0, NB_PH // NBUF, outer, 0)

    plsc.subcore_barrier()
    pltpu.sync_copy(acc.at[pl.ds(RPT * s, RPT)],
                    out_hbm.at[c, pl.ds(RPT * s, RPT)])


@functools.cache
def _agg_kernel():
    return pl.kernel(
    _agg_body,
    out_type=jax.ShapeDtypeStruct((NC, NACC, D), jnp.float32),
    mesh=_mesh(),
    compiler_params=pltpu.CompilerParams(needs_layout_passes=False),
    scratch_types=[
        pltpu.VMEM_SHARED((NACC, D), jnp.float32),
        pltpu.VMEM((NB_PH, K), jnp.int32),
        pltpu.VMEM((NB_PH, K), jnp.int32),
        [pltpu.VMEM((K, D), jnp.float32) for _ in range(NBUF)],
        [pltpu.SemaphoreType.DMA for _ in range(NBUF)],
    ],
    )


def _dinv(deg_ref):
    return lax.rsqrt(deg_ref[...] + 1.0)


def _mm_scale_body(x_ref, w_ref, deg_ref, o_ref):
    xw = jnp.dot(x_ref[...], w_ref[...], preferred_element_type=jnp.float32)
    o_ref[...] = xw * _dinv(deg_ref)


def _mm_scale(x, w, dinv):
    return pl.pallas_call(
        _mm_scale_body,
        grid=(N // ROWB,),
        in_specs=[
            pl.BlockSpec((ROWB, D), lambda i: (i, 0)),
            pl.BlockSpec((D, D), lambda i: (0, 0)),
            pl.BlockSpec((ROWB, D), lambda i: (i, 0)),
        ],
        out_specs=pl.BlockSpec((ROWB, D), lambda i: (i, 0)),
        out_shape=jax.ShapeDtypeStruct((N, D), jnp.float32),
    )(x, w, dinv)


def _mid_body(p_ref, y_ref, deg_ref, b_ref, w_ref, o_ref):
    dinv = _dinv(deg_ref)
    agg = p_ref[0] + p_ref[1] + y_ref[...]
    h = jnp.maximum(agg * dinv + b_ref[...], 0.0)
    hw = jnp.dot(h, w_ref[...], preferred_element_type=jnp.float32)
    o_ref[...] = hw * dinv


def _mid(partials, y, dinv, b1, w2):
    return pl.pallas_call(
        _mid_body,
        grid=(N // ROWB,),
        in_specs=[
            pl.BlockSpec((NC, ROWB, D), lambda i: (0, i, 0)),
            pl.BlockSpec((ROWB, D), lambda i: (i, 0)),
            pl.BlockSpec((ROWB, D), lambda i: (i, 0)),
            pl.BlockSpec((1, D), lambda i: (0, 0)),
            pl.BlockSpec((D, D), lambda i: (0, 0)),
        ],
        out_specs=pl.BlockSpec((ROWB, D), lambda i: (i, 0)),
        out_shape=jax.ShapeDtypeStruct((N, D), jnp.float32),
    )(partials, y, dinv, b1, w2)


def _final_body(p_ref, y_ref, deg_ref, b_ref, o_ref):
    agg = p_ref[0] + p_ref[1] + y_ref[...]
    o_ref[...] = agg * _dinv(deg_ref) + b_ref[...]


def _final(partials, y, dinv, b2):
    return pl.pallas_call(
        _final_body,
        grid=(N // ROWB,),
        in_specs=[
            pl.BlockSpec((NC, ROWB, D), lambda i: (0, i, 0)),
            pl.BlockSpec((ROWB, D), lambda i: (i, 0)),
            pl.BlockSpec((ROWB, D), lambda i: (i, 0)),
            pl.BlockSpec((1, D), lambda i: (0, 0)),
        ],
        out_specs=pl.BlockSpec((ROWB, D), lambda i: (i, 0)),
        out_shape=jax.ShapeDtypeStruct((N, D), jnp.float32),
    )(partials, y, dinv, b2)


def kernel(x, edge_index, W1, b1, W2, b2):
    ei = edge_index.astype(jnp.int32)
    pad = E_PAD - E
    fill = jnp.arange(pad, dtype=jnp.int32)
    src = jnp.concatenate([ei[0], fill % N])
    dst = jnp.concatenate([ei[1], N + (fill % (NACC - N))])
    src3 = src.reshape(NW, NB_AGG, K)
    dst3 = dst.reshape(NW, NB_AGG, K)
    dst3_deg = dst.reshape(NS, NB_DEG, K)
    zeros = jnp.zeros((NACC, D), jnp.float32)

    deg = _deg_kernel()(dst3_deg)
    deg10k = lax.slice(deg, (0, 0), (N, D))

    y1 = _mm_scale(x, W1, deg10k)
    p1 = _agg_kernel()(y1, src3, dst3, zeros)
    y2 = _mid(p1, y1, deg10k, b1.reshape(1, D), W2)
    p2 = _agg_kernel()(y2, src3, dst3, zeros)
    return _final(p2, y2, deg10k, b2.reshape(1, D))
